# Initial kernel scaffold; baseline (speedup 1.0000x reference)
import jax
import jax.numpy as jnp
from jax.experimental import pallas as pl


def kernel(x_author, x_paper, edge_index_writes, edge_index_rev, params):
    raise NotImplementedError("write your pallas kernel here")


# speedup vs baseline: 8.4001x; 8.4001x over previous
import jax
import jax.numpy as jnp
from jax import lax
from jax.experimental import pallas as pl
from jax.experimental.pallas import tpu as pltpu
from jax.experimental.pallas import tpu_sc as plsc

HID = 256
HHALF = 128
N_REAL = 5000
NPAD = 5120
E_REAL = 80000
EPAD = 81920
NBLK = 40
BLK = 128
ROWS_PER_TILE = NPAD // 16


_BM = 512


def _mm_body(x_ref, w_ref, b_ref, o_ref):
    o_ref[...] = (
        jnp.dot(x_ref[...], w_ref[...], preferred_element_type=jnp.float32)
        + b_ref[...]
    )


def _mm(x, w, b):
    m = x.shape[0]
    return pl.pallas_call(
        _mm_body,
        grid=(m // _BM,),
        in_specs=[
            pl.BlockSpec((_BM, HID), lambda i: (i, 0)),
            pl.BlockSpec((HID, HID), lambda i: (0, 0)),
            pl.BlockSpec((1, HID), lambda i: (0, 0)),
        ],
        out_specs=pl.BlockSpec((_BM, HID), lambda i: (i, 0)),
        out_shape=jax.ShapeDtypeStruct((m, HID), jnp.float32),
    )(x, w, b)


def _fin_body(xl_ref, xh_ref, w_ref, b_ref, o_ref):
    x = jnp.concatenate([xl_ref[...], xh_ref[...]], axis=1)
    o_ref[...] = (
        jnp.dot(x, w_ref[...], preferred_element_type=jnp.float32) + b_ref[...]
    )


def _fin(xl, xh, w, b):
    m = xl.shape[0]
    hspec = pl.BlockSpec((_BM, HHALF), lambda i: (i, 0))
    return pl.pallas_call(
        _fin_body,
        grid=(m // _BM,),
        in_specs=[
            hspec,
            hspec,
            pl.BlockSpec((HID, HID), lambda i: (0, 0)),
            pl.BlockSpec((1, HID), lambda i: (0, 0)),
        ],
        out_specs=pl.BlockSpec((_BM, HID), lambda i: (i, 0)),
        out_shape=jax.ShapeDtypeStruct((m, HID), jnp.float32),
    )(xl, xh, w, b)


def _kg_post(x, w_ref, b_ref, amt_ref, k_ref, g_ref):
    k = jnp.dot(x, w_ref[...], preferred_element_type=jnp.float32) + b_ref[...]
    k_ref[...] = k
    g_ref[...] = lax.dot_general(
        amt_ref[...], k, (((1,), (1,)), ((), ())),
        preferred_element_type=jnp.float32,
    )


def _kg_full_body(x_ref, w_ref, b_ref, amt_ref, k_ref, g_ref):
    _kg_post(x_ref[...], w_ref, b_ref, amt_ref, k_ref, g_ref)


def _kg_halves_body(xl_ref, xh_ref, w_ref, b_ref, amt_ref, k_ref, g_ref):
    x = jnp.concatenate([xl_ref[...], xh_ref[...]], axis=1)
    x = jnp.maximum(x, 0.01 * x)
    _kg_post(x, w_ref, b_ref, amt_ref, k_ref, g_ref)


def _kg_call(body, in_specs, args, m):
    return pl.pallas_call(
        body,
        grid=(m // _BM,),
        in_specs=in_specs + [
            pl.BlockSpec((HID, HID), lambda i: (0, 0)),
            pl.BlockSpec((1, HID), lambda i: (0, 0)),
            pl.BlockSpec((8, HID), lambda i: (0, 0)),
        ],
        out_specs=[
            pl.BlockSpec((_BM, HID), lambda i: (i, 0)),
            pl.BlockSpec((8, _BM), lambda i: (0, i)),
        ],
        out_shape=[
            jax.ShapeDtypeStruct((m, HID), jnp.float32),
            jax.ShapeDtypeStruct((8, m), jnp.float32),
        ],
    )(*args)


def _kg_full(x, w, b, amt):
    return _kg_call(
        _kg_full_body,
        [pl.BlockSpec((_BM, HID), lambda i: (i, 0))],
        [x, w, b, amt], x.shape[0])


def _kg_halves(xl, xh, w, b, amt):
    hspec = pl.BlockSpec((_BM, HHALF), lambda i: (i, 0))
    return _kg_call(
        _kg_halves_body, [hspec, hspec], [xl, xh, w, b, amt], xl.shape[0])


def _edge_body(ksrc_hbm, gsrc_hbm, gdst_hbm, srcp_hbm, dstp_hbm,
               z2_hbm, z1_hbm, out_hbm,
               src_v, dst_v, src2_v, asr_v, adst_v, ew_v, sden_v, w_v, rows_v,
               mxv_v, st_v, acc_sh, s_sh, mx_sh, sem):
    cid = lax.axis_index("c")
    sid = lax.axis_index("s")

    pltpu.sync_copy(srcp_hbm.at[pl.ds(sid * NBLK, NBLK)], src_v)
    pltpu.sync_copy(dstp_hbm.at[pl.ds(sid * NBLK, NBLK)], dst_v)
    pltpu.sync_copy(gsrc_hbm.at[0], asr_v)
    pltpu.sync_copy(gdst_hbm.at[1], adst_v)
    pltpu.sync_copy(z2_hbm, acc_sh.at[pl.ds(sid * ROWS_PER_TILE, ROWS_PER_TILE)])

    @pl.when(sid == 0)
    def _():
        pltpu.sync_copy(z1_hbm, s_sh)

    off = (cid * NPAD).astype(jnp.int32)

    def p1(j, cmax):
        for i in range(8):
            s16 = src_v[j, pl.ds(i * 16, 16)]
            d16 = dst_v[j, pl.ds(i * 16, 16)]
            src2_v[j, pl.ds(i * 16, 16)] = s16 + off
            av = plsc.load_gather(asr_v, [s16])
            bv = plsc.load_gather(adst_v, [d16])
            x = av + bv
            al = jnp.maximum(x, 0.2 * x)
            ew_v[j, pl.ds(i * 16, 16)] = al
            cmax = jnp.maximum(cmax, al)
        return cmax

    cmax = lax.fori_loop(0, NBLK, p1, jnp.full((16,), -1e30, jnp.float32))
    st_v[...] = jnp.full((16,), jnp.max(cmax), jnp.float32)
    pltpu.sync_copy(st_v, mx_sh.at[sid])
    plsc.subcore_barrier()

    pltpu.sync_copy(mx_sh, mxv_v)
    mv = mxv_v[0]
    for i in range(1, 16):
        mv = jnp.maximum(mv, mxv_v[i])

    def p2(j, carry):
        for i in range(8):
            al = ew_v[j, pl.ds(i * 16, 16)]
            ew_v[j, pl.ds(i * 16, 16)] = jnp.exp(al - mv)
        pltpu.sync_copy(ew_v.at[j], s_sh.at[dst_v.at[j]], add=True)
        return carry

    lax.fori_loop(0, NBLK, p2, 0)
    plsc.subcore_barrier()
    pltpu.sync_copy(s_sh, sden_v)

    def p3(j, carry):
        pltpu.async_copy(ksrc_hbm.at[src2_v.at[j]], rows_v, sem).wait()
        for i in range(8):
            d16 = dst_v[j, pl.ds(i * 16, 16)]
            e16 = ew_v[j, pl.ds(i * 16, 16)]
            sv = plsc.load_gather(sden_v, [d16])
            w_v[pl.ds(i * 16, 16)] = e16 / (sv + 1e-16)

        def scale_row(r, c2):
            wr = plsc.load_gather(w_v, [jnp.full((16,), 0, jnp.int32) + r])
            for c in range(8):
                rows_v[r, pl.ds(c * 16, 16)] = rows_v[r, pl.ds(c * 16, 16)] * wr
            return c2

        lax.fori_loop(0, BLK, scale_row, 0)
        pltpu.sync_copy(rows_v, acc_sh.at[dst_v.at[j]], add=True)
        return carry

    lax.fori_loop(0, NBLK, p3, 0)
    plsc.subcore_barrier()

    pltpu.sync_copy(
        acc_sh.at[pl.ds(sid * ROWS_PER_TILE, ROWS_PER_TILE)],
        out_hbm.at[cid, pl.ds(sid * ROWS_PER_TILE, ROWS_PER_TILE)],
    )


_edge_call = pl.kernel(
    _edge_body,
    out_type=jax.ShapeDtypeStruct((2, NPAD, HHALF), jnp.float32),
    mesh=plsc.VectorSubcoreMesh(core_axis_name="c", subcore_axis_name="s"),
    compiler_params=pltpu.CompilerParams(needs_layout_passes=False),
    scratch_types=[
        pltpu.VMEM((NBLK, BLK), jnp.int32),
        pltpu.VMEM((NBLK, BLK), jnp.int32),
        pltpu.VMEM((NBLK, BLK), jnp.int32),
        pltpu.VMEM((NPAD,), jnp.float32),
        pltpu.VMEM((NPAD,), jnp.float32),
        pltpu.VMEM((NBLK, BLK), jnp.float32),
        pltpu.VMEM((NPAD,), jnp.float32),
        pltpu.VMEM((BLK,), jnp.float32),
        pltpu.VMEM((BLK, HHALF), jnp.float32),
        pltpu.VMEM((16, 16), jnp.float32),
        pltpu.VMEM((16,), jnp.float32),
        pltpu.VMEM_SHARED((NPAD, HHALF), jnp.float32),
        pltpu.VMEM_SHARED((NPAD,), jnp.float32),
        pltpu.VMEM_SHARED((16, 16), jnp.float32),
        pltpu.SemaphoreType.DMA,
    ],
)


def _pad_edges(ei):
    s = jnp.pad(ei[0], (0, EPAD - ei.shape[1]), constant_values=N_REAL)
    d = jnp.pad(ei[1], (0, EPAD - ei.shape[1]), constant_values=N_REAL)
    return s.reshape(16 * NBLK, BLK), d.reshape(16 * NBLK, BLK)


def _split_k(k):
    return k.reshape(NPAD, 2, HHALF).transpose(1, 0, 2).reshape(2 * NPAD, HHALF)


def _amt(att_src_vec, att_dst_vec):
    z = jnp.zeros((6, HID), jnp.float32)
    return jnp.concatenate([att_src_vec[None], att_dst_vec[None], z], axis=0)


def kernel(x_author, x_paper, edge_index_writes, edge_index_rev, params):
    p = params
    xa = jnp.pad(x_author, ((0, NPAD - N_REAL), (0, 0)))
    xp = jnp.pad(x_paper, ((0, NPAD - N_REAL), (0, 0)))
    sw, dw = _pad_edges(edge_index_writes)
    sr, dr = _pad_edges(edge_index_rev)
    z2 = jnp.zeros((ROWS_PER_TILE, HHALF), jnp.float32)
    z1 = jnp.zeros((NPAD,), jnp.float32)

    ha = _mm(xa, p['Win']['author'], p['bin']['author'][None])
    hp = _mm(xp, p['Win']['paper'], p['bin']['paper'][None])

    def conv(c, kg_a, kg_p):
        amt_a = _amt(c['att_src']['writes'], c['att_dst']['rev'])
        amt_p = _amt(c['att_src']['rev'], c['att_dst']['writes'])
        ka, ga = kg_a(c['proj_W']['author'], c['proj_b']['author'][None], amt_a)
        kp, gp = kg_p(c['proj_W']['paper'], c['proj_b']['paper'][None], amt_p)
        out_p = _edge_call(_split_k(ka), ga, gp, sw, dw, z2, z1)
        out_a = _edge_call(_split_k(kp), gp, ga, sr, dr, z2, z1)
        return out_a, out_p

    oa1, op1 = conv(
        p['conv1'],
        lambda w, b, a: _kg_full(ha, w, b, a),
        lambda w, b, a: _kg_full(hp, w, b, a),
    )
    oa2, op2 = conv(
        p['conv2'],
        lambda w, b, a: _kg_halves(oa1[0], oa1[1], w, b, a),
        lambda w, b, a: _kg_halves(op1[0], op1[1], w, b, a),
    )

    out_a = _fin(oa2[0], oa2[1], p['Wout'], p['bout'][None])
    out_p = _fin(op2[0], op2[1], p['Wout'], p['bout'][None])
    return jnp.concatenate([out_a[:N_REAL], out_p[:N_REAL]], axis=0)


# speedup vs baseline: 11.3361x; 1.3495x over previous
import jax
import jax.numpy as jnp
from jax import lax
from jax.experimental import pallas as pl
from jax.experimental.pallas import tpu as pltpu
from jax.experimental.pallas import tpu_sc as plsc

HID = 256
HHALF = 128
N_REAL = 5000
NPAD = 5120
E_REAL = 80000
EPAD = 81920
NBLK = 40
BLK = 128
ROWS_PER_TILE = NPAD // 16


_BM = 512


def _mm_body(x_ref, w_ref, b_ref, o_ref):
    o_ref[...] = (
        jnp.dot(x_ref[...], w_ref[...], preferred_element_type=jnp.float32)
        + b_ref[...]
    )


def _mm(x, w, b):
    m = x.shape[0]
    return pl.pallas_call(
        _mm_body,
        grid=(m // _BM,),
        in_specs=[
            pl.BlockSpec((_BM, HID), lambda i: (i, 0)),
            pl.BlockSpec((HID, HID), lambda i: (0, 0)),
            pl.BlockSpec((1, HID), lambda i: (0, 0)),
        ],
        out_specs=pl.BlockSpec((_BM, HID), lambda i: (i, 0)),
        out_shape=jax.ShapeDtypeStruct((m, HID), jnp.float32),
    )(x, w, b)


def _fin_body(xl_ref, xh_ref, w_ref, b_ref, o_ref):
    x = jnp.concatenate([xl_ref[...], xh_ref[...]], axis=1)
    o_ref[...] = (
        jnp.dot(x, w_ref[...], preferred_element_type=jnp.float32) + b_ref[...]
    )


def _fin(xl, xh, w, b):
    m = xl.shape[0]
    hspec = pl.BlockSpec((_BM, HHALF), lambda i: (i, 0))
    return pl.pallas_call(
        _fin_body,
        grid=(m // _BM,),
        in_specs=[
            hspec,
            hspec,
            pl.BlockSpec((HID, HID), lambda i: (0, 0)),
            pl.BlockSpec((1, HID), lambda i: (0, 0)),
        ],
        out_specs=pl.BlockSpec((_BM, HID), lambda i: (i, 0)),
        out_shape=jax.ShapeDtypeStruct((m, HID), jnp.float32),
    )(xl, xh, w, b)


def _kg_post(x, w_ref, b_ref, amt_ref, k_ref, g_ref):
    k = jnp.dot(x, w_ref[...], preferred_element_type=jnp.float32) + b_ref[...]
    k_ref[...] = k
    g_ref[...] = lax.dot_general(
        amt_ref[...], k, (((1,), (1,)), ((), ())),
        preferred_element_type=jnp.float32,
    )


def _kg_full_body(x_ref, w_ref, b_ref, amt_ref, k_ref, g_ref):
    _kg_post(x_ref[...], w_ref, b_ref, amt_ref, k_ref, g_ref)


def _kg_halves_body(xl_ref, xh_ref, w_ref, b_ref, amt_ref, k_ref, g_ref):
    x = jnp.concatenate([xl_ref[...], xh_ref[...]], axis=1)
    x = jnp.maximum(x, 0.01 * x)
    _kg_post(x, w_ref, b_ref, amt_ref, k_ref, g_ref)


def _kg_call(body, in_specs, args, m):
    return pl.pallas_call(
        body,
        grid=(m // _BM,),
        in_specs=in_specs + [
            pl.BlockSpec((HID, HID), lambda i: (0, 0)),
            pl.BlockSpec((1, HID), lambda i: (0, 0)),
            pl.BlockSpec((8, HID), lambda i: (0, 0)),
        ],
        out_specs=[
            pl.BlockSpec((_BM, HID), lambda i: (i, 0)),
            pl.BlockSpec((8, _BM), lambda i: (0, i)),
        ],
        out_shape=[
            jax.ShapeDtypeStruct((m, HID), jnp.float32),
            jax.ShapeDtypeStruct((8, m), jnp.float32),
        ],
    )(*args)


def _kg_full(x, w, b, amt):
    return _kg_call(
        _kg_full_body,
        [pl.BlockSpec((_BM, HID), lambda i: (i, 0))],
        [x, w, b, amt], x.shape[0])


def _kg_halves(xl, xh, w, b, amt):
    hspec = pl.BlockSpec((_BM, HHALF), lambda i: (i, 0))
    return _kg_call(
        _kg_halves_body, [hspec, hspec], [xl, xh, w, b, amt], xl.shape[0])


def _edge_body(ksrc_hbm, gsrc_hbm, gdst_hbm, srcp_hbm, dstp_hbm,
               z2_hbm, z1_hbm, out_hbm,
               src_v, dst_v, src2_v, asr_v, adst_v, ew_v, sden_v, w_v, rows_v,
               mxv_v, st_v, acc_sh, s_sh, mx_sh, sem, sem_s):
    cid = lax.axis_index("c")
    sid = lax.axis_index("s")

    pltpu.sync_copy(srcp_hbm.at[pl.ds(sid * NBLK, NBLK)], src_v)
    pltpu.sync_copy(dstp_hbm.at[pl.ds(sid * NBLK, NBLK)], dst_v)
    pltpu.sync_copy(gsrc_hbm.at[0], asr_v)
    pltpu.sync_copy(gdst_hbm.at[1], adst_v)
    pltpu.sync_copy(z2_hbm, acc_sh.at[pl.ds(sid * ROWS_PER_TILE, ROWS_PER_TILE)])

    @pl.when(sid == 0)
    def _():
        pltpu.sync_copy(z1_hbm, s_sh)

    off = (cid * NPAD).astype(jnp.int32)

    def p1(j, cmax):
        for i in range(8):
            s16 = src_v[j, pl.ds(i * 16, 16)]
            d16 = dst_v[j, pl.ds(i * 16, 16)]
            src2_v[j, pl.ds(i * 16, 16)] = s16 + off
            av = plsc.load_gather(asr_v, [s16])
            bv = plsc.load_gather(adst_v, [d16])
            x = av + bv
            al = jnp.maximum(x, 0.2 * x)
            ew_v[j, pl.ds(i * 16, 16)] = al
            cmax = jnp.maximum(cmax, al)
        return cmax

    cmax = lax.fori_loop(0, NBLK, p1, jnp.full((16,), -1e30, jnp.float32))
    st_v[...] = jnp.full((16,), jnp.max(cmax), jnp.float32)
    pltpu.sync_copy(st_v, mx_sh.at[sid])
    plsc.subcore_barrier()

    pltpu.sync_copy(mx_sh, mxv_v)
    mv = mxv_v[0]
    for i in range(1, 16):
        mv = jnp.maximum(mv, mxv_v[i])

    def p2(jc, carry):
        for i8 in range(8):
            j = jc * 8 + i8
            for i in range(8):
                al = ew_v[j, pl.ds(i * 16, 16)]
                ew_v[j, pl.ds(i * 16, 16)] = jnp.exp(al - mv)
            pltpu.async_copy(ew_v.at[j], s_sh.at[dst_v.at[j]], sem_s, add=True)
        for i8 in range(8):
            j = jc * 8 + i8
            pltpu.make_async_copy(
                ew_v.at[j], s_sh.at[dst_v.at[j]], sem_s).wait()
        return carry

    lax.fori_loop(0, NBLK // 8, p2, 0)
    plsc.subcore_barrier()
    pltpu.sync_copy(s_sh, sden_v)

    def g_start(j, b):
        pltpu.async_copy(ksrc_hbm.at[src2_v.at[j]], rows_v.at[b], sem)

    def g_wait(j, b):
        pltpu.make_async_copy(ksrc_hbm.at[src2_v.at[j]], rows_v.at[b],
                              sem).wait()

    def s_start(j, b):
        pltpu.async_copy(rows_v.at[b], acc_sh.at[dst_v.at[j]], sem_s,
                         add=True)

    def s_wait(j, b):
        pltpu.make_async_copy(rows_v.at[b], acc_sh.at[dst_v.at[j]],
                              sem_s).wait()

    g_start(0, 0)

    def p3(jj, carry):
        for b in range(2):
            j = jj * 2 + b
            g_wait(j, b)

            @pl.when(j + 1 < NBLK)
            def _():
                @pl.when(j >= 1)
                def _():
                    s_wait(j - 1, 1 - b)
                g_start(j + 1, 1 - b)

            for i in range(8):
                d16 = dst_v[j, pl.ds(i * 16, 16)]
                e16 = ew_v[j, pl.ds(i * 16, 16)]
                sv = plsc.load_gather(sden_v, [d16])
                w_v[pl.ds(i * 16, 16)] = e16 / (sv + 1e-16)

            @plsc.parallel_loop(0, BLK)
            def _(r):
                wr = plsc.load_gather(w_v, [jnp.full((16,), 0, jnp.int32) + r])
                for c in range(8):
                    rv = rows_v[b, r, pl.ds(c * 16, 16)]
                    rows_v[b, r, pl.ds(c * 16, 16)] = rv * wr

            s_start(j, b)
        return carry

    lax.fori_loop(0, NBLK // 2, p3, 0)
    s_wait(NBLK - 2, 0)
    s_wait(NBLK - 1, 1)
    plsc.subcore_barrier()

    pltpu.sync_copy(
        acc_sh.at[pl.ds(sid * ROWS_PER_TILE, ROWS_PER_TILE)],
        out_hbm.at[cid, pl.ds(sid * ROWS_PER_TILE, ROWS_PER_TILE)],
    )


_edge_call = pl.kernel(
    _edge_body,
    out_type=jax.ShapeDtypeStruct((2, NPAD, HHALF), jnp.float32),
    mesh=plsc.VectorSubcoreMesh(core_axis_name="c", subcore_axis_name="s"),
    compiler_params=pltpu.CompilerParams(needs_layout_passes=False),
    scratch_types=[
        pltpu.VMEM((NBLK, BLK), jnp.int32),
        pltpu.VMEM((NBLK, BLK), jnp.int32),
        pltpu.VMEM((NBLK, BLK), jnp.int32),
        pltpu.VMEM((NPAD,), jnp.float32),
        pltpu.VMEM((NPAD,), jnp.float32),
        pltpu.VMEM((NBLK, BLK), jnp.float32),
        pltpu.VMEM((NPAD,), jnp.float32),
        pltpu.VMEM((BLK,), jnp.float32),
        pltpu.VMEM((2, BLK, HHALF), jnp.float32),
        pltpu.VMEM((16, 16), jnp.float32),
        pltpu.VMEM((16,), jnp.float32),
        pltpu.VMEM_SHARED((NPAD, HHALF), jnp.float32),
        pltpu.VMEM_SHARED((NPAD,), jnp.float32),
        pltpu.VMEM_SHARED((16, 16), jnp.float32),
        pltpu.SemaphoreType.DMA,
        pltpu.SemaphoreType.DMA,
    ],
)


def _pad_edges(ei):
    s = jnp.pad(ei[0], (0, EPAD - ei.shape[1]), constant_values=N_REAL)
    d = jnp.pad(ei[1], (0, EPAD - ei.shape[1]), constant_values=N_REAL)
    return s.reshape(16 * NBLK, BLK), d.reshape(16 * NBLK, BLK)


def _split_k(k):
    return k.reshape(NPAD, 2, HHALF).transpose(1, 0, 2).reshape(2 * NPAD, HHALF)


def _amt(att_src_vec, att_dst_vec):
    z = jnp.zeros((6, HID), jnp.float32)
    return jnp.concatenate([att_src_vec[None], att_dst_vec[None], z], axis=0)


def kernel(x_author, x_paper, edge_index_writes, edge_index_rev, params):
    p = params
    xa = jnp.pad(x_author, ((0, NPAD - N_REAL), (0, 0)))
    xp = jnp.pad(x_paper, ((0, NPAD - N_REAL), (0, 0)))
    sw, dw = _pad_edges(edge_index_writes)
    sr, dr = _pad_edges(edge_index_rev)
    z2 = jnp.zeros((ROWS_PER_TILE, HHALF), jnp.float32)
    z1 = jnp.zeros((NPAD,), jnp.float32)

    ha = _mm(xa, p['Win']['author'], p['bin']['author'][None])
    hp = _mm(xp, p['Win']['paper'], p['bin']['paper'][None])

    def conv(c, kg_a, kg_p):
        amt_a = _amt(c['att_src']['writes'], c['att_dst']['rev'])
        amt_p = _amt(c['att_src']['rev'], c['att_dst']['writes'])
        ka, ga = kg_a(c['proj_W']['author'], c['proj_b']['author'][None], amt_a)
        kp, gp = kg_p(c['proj_W']['paper'], c['proj_b']['paper'][None], amt_p)
        out_p = _edge_call(_split_k(ka), ga, gp, sw, dw, z2, z1)
        out_a = _edge_call(_split_k(kp), gp, ga, sr, dr, z2, z1)
        return out_a, out_p

    oa1, op1 = conv(
        p['conv1'],
        lambda w, b, a: _kg_full(ha, w, b, a),
        lambda w, b, a: _kg_full(hp, w, b, a),
    )
    oa2, op2 = conv(
        p['conv2'],
        lambda w, b, a: _kg_halves(oa1[0], oa1[1], w, b, a),
        lambda w, b, a: _kg_halves(op1[0], op1[1], w, b, a),
    )

    out_a = _fin(oa2[0], oa2[1], p['Wout'], p['bout'][None])
    out_p = _fin(op2[0], op2[1], p['Wout'], p['bout'][None])
    return jnp.concatenate([out_a[:N_REAL], out_p[:N_REAL]], axis=0)


# speedup vs baseline: 21.5367x; 1.8998x over previous
import jax
import jax.numpy as jnp
from jax import lax
from jax.experimental import pallas as pl
from jax.experimental.pallas import tpu as pltpu
from jax.experimental.pallas import tpu_sc as plsc

HID = 256
HHALF = 128
N_REAL = 5000
NPAD = 5120
E_REAL = 80000
EPAD = 81920
NBLK = 40
BLK = 128
ROWS_PER_TILE = NPAD // 16


_BM = 512


def _mm_body(x_ref, w_ref, b_ref, o_ref):
    o_ref[...] = (
        jnp.dot(x_ref[...], w_ref[...], preferred_element_type=jnp.float32)
        + b_ref[...]
    )


def _mm(x, w, b):
    m = x.shape[0]
    return pl.pallas_call(
        _mm_body,
        grid=(m // _BM,),
        in_specs=[
            pl.BlockSpec((_BM, HID), lambda i: (i, 0)),
            pl.BlockSpec((HID, HID), lambda i: (0, 0)),
            pl.BlockSpec((1, HID), lambda i: (0, 0)),
        ],
        out_specs=pl.BlockSpec((_BM, HID), lambda i: (i, 0)),
        out_shape=jax.ShapeDtypeStruct((m, HID), jnp.float32),
    )(x, w, b)


def _fin_body(xl_ref, xh_ref, w_ref, b_ref, o_ref):
    x = jnp.concatenate([xl_ref[...], xh_ref[...]], axis=1)
    o_ref[...] = (
        jnp.dot(x, w_ref[...], preferred_element_type=jnp.float32) + b_ref[...]
    )


def _fin(xl, xh, w, b):
    m = xl.shape[0]
    hspec = pl.BlockSpec((_BM, HHALF), lambda i: (i, 0))
    return pl.pallas_call(
        _fin_body,
        grid=(m // _BM,),
        in_specs=[
            hspec,
            hspec,
            pl.BlockSpec((HID, HID), lambda i: (0, 0)),
            pl.BlockSpec((1, HID), lambda i: (0, 0)),
        ],
        out_specs=pl.BlockSpec((_BM, HID), lambda i: (i, 0)),
        out_shape=jax.ShapeDtypeStruct((m, HID), jnp.float32),
    )(xl, xh, w, b)


def _kg_post(x, w_ref, b_ref, amt_ref, k_ref, g_ref):
    k = jnp.dot(x, w_ref[...], preferred_element_type=jnp.float32) + b_ref[...]
    k_ref[...] = k
    g_ref[...] = lax.dot_general(
        amt_ref[...], k, (((1,), (1,)), ((), ())),
        preferred_element_type=jnp.float32,
    )


def _kg_full_body(x_ref, w_ref, b_ref, amt_ref, k_ref, g_ref):
    _kg_post(x_ref[...], w_ref, b_ref, amt_ref, k_ref, g_ref)


def _kg_halves_body(xl_ref, xh_ref, w_ref, b_ref, amt_ref, k_ref, g_ref):
    x = jnp.concatenate([xl_ref[...], xh_ref[...]], axis=1)
    x = jnp.maximum(x, 0.01 * x)
    _kg_post(x, w_ref, b_ref, amt_ref, k_ref, g_ref)


def _kg_call(body, in_specs, args, m):
    return pl.pallas_call(
        body,
        grid=(m // _BM,),
        in_specs=in_specs + [
            pl.BlockSpec((HID, HID), lambda i: (0, 0)),
            pl.BlockSpec((1, HID), lambda i: (0, 0)),
            pl.BlockSpec((8, HID), lambda i: (0, 0)),
        ],
        out_specs=[
            pl.BlockSpec((_BM, HID), lambda i: (i, 0)),
            pl.BlockSpec((8, _BM), lambda i: (0, i)),
        ],
        out_shape=[
            jax.ShapeDtypeStruct((m, HID), jnp.float32),
            jax.ShapeDtypeStruct((8, m), jnp.float32),
        ],
    )(*args)


def _kg_full(x, w, b, amt):
    return _kg_call(
        _kg_full_body,
        [pl.BlockSpec((_BM, HID), lambda i: (i, 0))],
        [x, w, b, amt], x.shape[0])


def _kg_halves(xl, xh, w, b, amt):
    hspec = pl.BlockSpec((_BM, HHALF), lambda i: (i, 0))
    return _kg_call(
        _kg_halves_body, [hspec, hspec], [xl, xh, w, b, amt], xl.shape[0])


def _edge_body(ksrc_hbm, gsrc_hbm, gdst_hbm, srcp_hbm, dstp_hbm,
               out_hbm,
               src_v, dst_v, src2_v, asr_v, adst_v, ew_v, sden_v, w_v, rows_v,
               zb_v, mxv_v, st_v, acc_sh, s_sh, mx_sh, sem, sem_s):
    cid = lax.axis_index("c")
    sid = lax.axis_index("s")

    pltpu.async_copy(srcp_hbm.at[pl.ds(sid * NBLK, NBLK)], src_v, sem)
    pltpu.async_copy(dstp_hbm.at[pl.ds(sid * NBLK, NBLK)], dst_v, sem)
    pltpu.async_copy(gsrc_hbm.at[0], asr_v, sem)
    pltpu.async_copy(gdst_hbm.at[1], adst_v, sem)

    zv = jnp.zeros((16,), jnp.float32)

    def zb(i, c2):
        for k in range(8):
            zb_v[i, pl.ds(k * 16, 16)] = zv
        return c2

    lax.fori_loop(0, 64, zb, 0)
    for t in range(ROWS_PER_TILE // 64):
        pltpu.async_copy(
            zb_v, acc_sh.at[pl.ds(sid * ROWS_PER_TILE + t * 64, 64)], sem_s)

    pltpu.make_async_copy(srcp_hbm.at[pl.ds(0, NBLK)], src_v, sem).wait()
    pltpu.make_async_copy(dstp_hbm.at[pl.ds(0, NBLK)], dst_v, sem).wait()
    pltpu.make_async_copy(gsrc_hbm.at[0], asr_v, sem).wait()
    pltpu.make_async_copy(gdst_hbm.at[1], adst_v, sem).wait()
    for t in range(ROWS_PER_TILE // 64):
        pltpu.make_async_copy(
            zb_v, acc_sh.at[pl.ds(sid * ROWS_PER_TILE + t * 64, 64)],
            sem_s).wait()

    def zs(i, c2):
        sden_v[pl.ds(i * 16, 16)] = zv
        return c2

    lax.fori_loop(0, NPAD // 16, zs, 0)

    @pl.when(sid == 0)
    def _():
        pltpu.sync_copy(sden_v, s_sh)

    off = (cid * NPAD).astype(jnp.int32)

    def p1(j, cmax):
        for i in range(8):
            s16 = src_v[j, pl.ds(i * 16, 16)]
            d16 = dst_v[j, pl.ds(i * 16, 16)]
            src2_v[j, pl.ds(i * 16, 16)] = s16 + off
            av = plsc.load_gather(asr_v, [s16])
            bv = plsc.load_gather(adst_v, [d16])
            x = av + bv
            al = jnp.maximum(x, 0.2 * x)
            ew_v[j, pl.ds(i * 16, 16)] = al
            cmax = jnp.maximum(cmax, al)
        return cmax

    cmax = lax.fori_loop(0, NBLK, p1, jnp.full((16,), -1e30, jnp.float32))
    st_v[...] = jnp.full((16,), jnp.max(cmax), jnp.float32)
    pltpu.sync_copy(st_v, mx_sh.at[sid])
    plsc.subcore_barrier()

    pltpu.sync_copy(mx_sh, mxv_v)
    mv = mxv_v[0]
    for i in range(1, 16):
        mv = jnp.maximum(mv, mxv_v[i])

    def p2(jc, carry):
        for i8 in range(8):
            j = jc * 8 + i8
            for i in range(8):
                al = ew_v[j, pl.ds(i * 16, 16)]
                ew_v[j, pl.ds(i * 16, 16)] = jnp.exp(al - mv)
            pltpu.async_copy(ew_v.at[j], s_sh.at[dst_v.at[j]], sem_s, add=True)
        for i8 in range(8):
            j = jc * 8 + i8
            pltpu.make_async_copy(
                ew_v.at[j], s_sh.at[dst_v.at[j]], sem_s).wait()
        return carry

    lax.fori_loop(0, NBLK // 8, p2, 0)
    plsc.subcore_barrier()
    pltpu.sync_copy(s_sh, sden_v)

    def g_start(j, b):
        pltpu.async_copy(ksrc_hbm.at[src2_v.at[j]], rows_v.at[b], sem)

    def g_wait(j, b):
        pltpu.make_async_copy(ksrc_hbm.at[src2_v.at[j]], rows_v.at[b],
                              sem).wait()

    def s_start(j, b):
        pltpu.async_copy(rows_v.at[b], acc_sh.at[dst_v.at[j]], sem_s,
                         add=True)

    def s_wait(j, b):
        pltpu.make_async_copy(rows_v.at[b], acc_sh.at[dst_v.at[j]],
                              sem_s).wait()

    g_start(0, 0)

    def p3(jj, carry):
        for b in range(2):
            j = jj * 2 + b
            g_wait(j, b)

            @pl.when(j + 1 < NBLK)
            def _():
                @pl.when(j >= 1)
                def _():
                    s_wait(j - 1, 1 - b)
                g_start(j + 1, 1 - b)

            for i in range(8):
                d16 = dst_v[j, pl.ds(i * 16, 16)]
                e16 = ew_v[j, pl.ds(i * 16, 16)]
                sv = plsc.load_gather(sden_v, [d16])
                w_v[pl.ds(i * 16, 16)] = e16 / (sv + 1e-16)

            @plsc.parallel_loop(0, BLK)
            def _(r):
                wr = plsc.load_gather(w_v, [jnp.full((16,), 0, jnp.int32) + r])
                for c in range(8):
                    rv = rows_v[b, r, pl.ds(c * 16, 16)]
                    rows_v[b, r, pl.ds(c * 16, 16)] = rv * wr

            s_start(j, b)
        return carry

    lax.fori_loop(0, NBLK // 2, p3, 0)
    s_wait(NBLK - 2, 0)
    s_wait(NBLK - 1, 1)
    plsc.subcore_barrier()

    pltpu.sync_copy(
        acc_sh.at[pl.ds(sid * ROWS_PER_TILE, ROWS_PER_TILE)],
        out_hbm.at[cid, pl.ds(sid * ROWS_PER_TILE, ROWS_PER_TILE)],
    )


_edge_call = pl.kernel(
    _edge_body,
    out_type=jax.ShapeDtypeStruct((2, NPAD, HHALF), jnp.float32),
    mesh=plsc.VectorSubcoreMesh(core_axis_name="c", subcore_axis_name="s"),
    compiler_params=pltpu.CompilerParams(needs_layout_passes=False),
    scratch_types=[
        pltpu.VMEM((NBLK, BLK), jnp.int32),
        pltpu.VMEM((NBLK, BLK), jnp.int32),
        pltpu.VMEM((NBLK, BLK), jnp.int32),
        pltpu.VMEM((NPAD,), jnp.float32),
        pltpu.VMEM((NPAD,), jnp.float32),
        pltpu.VMEM((NBLK, BLK), jnp.float32),
        pltpu.VMEM((NPAD,), jnp.float32),
        pltpu.VMEM((BLK,), jnp.float32),
        pltpu.VMEM((2, BLK, HHALF), jnp.float32),
        pltpu.VMEM((64, HHALF), jnp.float32),
        pltpu.VMEM((16, 16), jnp.float32),
        pltpu.VMEM((16,), jnp.float32),
        pltpu.VMEM_SHARED((NPAD, HHALF), jnp.float32),
        pltpu.VMEM_SHARED((NPAD,), jnp.float32),
        pltpu.VMEM_SHARED((16, 16), jnp.float32),
        pltpu.SemaphoreType.DMA,
        pltpu.SemaphoreType.DMA,
    ],
)


def _pad_edges(ei):
    npad = EPAD - ei.shape[1]
    fill = N_REAL + (jnp.arange(npad, dtype=jnp.int32) % (NPAD - N_REAL))
    s = jnp.concatenate([ei[0], fill])
    d = jnp.concatenate([ei[1], fill])
    return s.reshape(16 * NBLK, BLK), d.reshape(16 * NBLK, BLK)


def _split_k(k):
    return k.reshape(NPAD, 2, HHALF).transpose(1, 0, 2).reshape(2 * NPAD, HHALF)


def _amt(att_src_vec, att_dst_vec):
    z = jnp.zeros((6, HID), jnp.float32)
    return jnp.concatenate([att_src_vec[None], att_dst_vec[None], z], axis=0)


def kernel(x_author, x_paper, edge_index_writes, edge_index_rev, params):
    p = params
    xa = jnp.pad(x_author, ((0, NPAD - N_REAL), (0, 0)))
    xp = jnp.pad(x_paper, ((0, NPAD - N_REAL), (0, 0)))
    sw, dw = _pad_edges(edge_index_writes)
    sr, dr = _pad_edges(edge_index_rev)

    ha = _mm(xa, p['Win']['author'], p['bin']['author'][None])
    hp = _mm(xp, p['Win']['paper'], p['bin']['paper'][None])

    def conv(c, kg_a, kg_p):
        amt_a = _amt(c['att_src']['writes'], c['att_dst']['rev'])
        amt_p = _amt(c['att_src']['rev'], c['att_dst']['writes'])
        ka, ga = kg_a(c['proj_W']['author'], c['proj_b']['author'][None], amt_a)
        kp, gp = kg_p(c['proj_W']['paper'], c['proj_b']['paper'][None], amt_p)
        out_p = _edge_call(_split_k(ka), ga, gp, sw, dw)
        out_a = _edge_call(_split_k(kp), gp, ga, sr, dr)
        return out_a, out_p

    oa1, op1 = conv(
        p['conv1'],
        lambda w, b, a: _kg_full(ha, w, b, a),
        lambda w, b, a: _kg_full(hp, w, b, a),
    )
    oa2, op2 = conv(
        p['conv2'],
        lambda w, b, a: _kg_halves(oa1[0], oa1[1], w, b, a),
        lambda w, b, a: _kg_halves(op1[0], op1[1], w, b, a),
    )

    out_a = _fin(oa2[0], oa2[1], p['Wout'], p['bout'][None])
    out_p = _fin(op2[0], op2[1], p['Wout'], p['bout'][None])
    return jnp.concatenate([out_a[:N_REAL], out_p[:N_REAL]], axis=0)


# speedup vs baseline: 22.0510x; 1.0239x over previous
import jax
import jax.numpy as jnp
from jax import lax
from jax.experimental import pallas as pl
from jax.experimental.pallas import tpu as pltpu
from jax.experimental.pallas import tpu_sc as plsc

HID = 256
HHALF = 128
N_REAL = 5000
NPAD = 5120
E_REAL = 80000
EPAD = 81920
NBLK = 40
BLK = 128
ROWS_PER_TILE = NPAD // 16


_BM = 512


def _fin_body(xl_ref, xh_ref, w_ref, b_ref, o_ref):
    x = jnp.concatenate([xl_ref[...], xh_ref[...]], axis=1)
    o_ref[...] = (
        jnp.dot(x, w_ref[...], preferred_element_type=jnp.float32) + b_ref[...]
    )


def _fin(xl, xh, w, b):
    m = xl.shape[0]
    hspec = pl.BlockSpec((_BM, HHALF), lambda i: (i, 0))
    return pl.pallas_call(
        _fin_body,
        grid=(m // _BM,),
        in_specs=[
            hspec,
            hspec,
            pl.BlockSpec((HID, HID), lambda i: (0, 0)),
            pl.BlockSpec((1, HID), lambda i: (0, 0)),
        ],
        out_specs=pl.BlockSpec((_BM, HID), lambda i: (i, 0)),
        out_shape=jax.ShapeDtypeStruct((m, HID), jnp.float32),
    )(xl, xh, w, b)


def _kg_post(x, w_ref, b_ref, amt_ref, k_ref, g_ref):
    k = jnp.dot(x, w_ref[...], preferred_element_type=jnp.float32) + b_ref[...]
    k_ref[...] = k
    g_ref[...] = lax.dot_general(
        amt_ref[...], k, (((1,), (1,)), ((), ())),
        preferred_element_type=jnp.float32,
    )


def _kg_full_body(x_ref, w_ref, b_ref, amt_ref, k_ref, g_ref):
    _kg_post(x_ref[...], w_ref, b_ref, amt_ref, k_ref, g_ref)


def _kg_halves_body(xl_ref, xh_ref, w_ref, b_ref, amt_ref, k_ref, g_ref):
    x = jnp.concatenate([xl_ref[...], xh_ref[...]], axis=1)
    x = jnp.maximum(x, 0.01 * x)
    _kg_post(x, w_ref, b_ref, amt_ref, k_ref, g_ref)


def _kg_call(body, in_specs, args, m):
    return pl.pallas_call(
        body,
        grid=(m // _BM,),
        in_specs=in_specs + [
            pl.BlockSpec((HID, HID), lambda i: (0, 0)),
            pl.BlockSpec((1, HID), lambda i: (0, 0)),
            pl.BlockSpec((8, HID), lambda i: (0, 0)),
        ],
        out_specs=[
            pl.BlockSpec((_BM, HID), lambda i: (i, 0)),
            pl.BlockSpec((8, _BM), lambda i: (0, i)),
        ],
        out_shape=[
            jax.ShapeDtypeStruct((m, HID), jnp.float32),
            jax.ShapeDtypeStruct((8, m), jnp.float32),
        ],
    )(*args)


def _kg_full(x, w, b, amt):
    return _kg_call(
        _kg_full_body,
        [pl.BlockSpec((_BM, HID), lambda i: (i, 0))],
        [x, w, b, amt], x.shape[0])


def _kg_halves(xl, xh, w, b, amt):
    hspec = pl.BlockSpec((_BM, HHALF), lambda i: (i, 0))
    return _kg_call(
        _kg_halves_body, [hspec, hspec], [xl, xh, w, b, amt], xl.shape[0])


def _edge_body(ksrc_hbm, gsrc_hbm, gdst_hbm, srcp_hbm, dstp_hbm,
               out_hbm,
               src_v, dst_v, src2_v, asr_v, adst_v, ew_v, sden_v, w_v, rows_v,
               zb_v, mxv_v, st_v, acc_sh, s_sh, mx_sh, sem, sem_s):
    cid = lax.axis_index("c")
    sid = lax.axis_index("s")

    pltpu.async_copy(srcp_hbm.at[pl.ds(sid * NBLK, NBLK)], src_v, sem)
    pltpu.async_copy(dstp_hbm.at[pl.ds(sid * NBLK, NBLK)], dst_v, sem)
    pltpu.async_copy(gsrc_hbm.at[0], asr_v, sem)
    pltpu.async_copy(gdst_hbm.at[1], adst_v, sem)

    zv = jnp.zeros((16,), jnp.float32)

    def zb(i, c2):
        for k in range(8):
            zb_v[i, pl.ds(k * 16, 16)] = zv
        return c2

    lax.fori_loop(0, 64, zb, 0)
    for t in range(ROWS_PER_TILE // 64):
        pltpu.async_copy(
            zb_v, acc_sh.at[pl.ds(sid * ROWS_PER_TILE + t * 64, 64)], sem_s)

    pltpu.make_async_copy(srcp_hbm.at[pl.ds(0, NBLK)], src_v, sem).wait()
    pltpu.make_async_copy(dstp_hbm.at[pl.ds(0, NBLK)], dst_v, sem).wait()
    pltpu.make_async_copy(gsrc_hbm.at[0], asr_v, sem).wait()
    pltpu.make_async_copy(gdst_hbm.at[1], adst_v, sem).wait()
    for t in range(ROWS_PER_TILE // 64):
        pltpu.make_async_copy(
            zb_v, acc_sh.at[pl.ds(sid * ROWS_PER_TILE + t * 64, 64)],
            sem_s).wait()

    def zs(i, c2):
        sden_v[pl.ds(i * 16, 16)] = zv
        return c2

    lax.fori_loop(0, NPAD // 16, zs, 0)

    @pl.when(sid == 0)
    def _():
        pltpu.sync_copy(sden_v, s_sh)

    off = (cid * NPAD).astype(jnp.int32)

    def p1(j, cmax):
        for i in range(8):
            s16 = src_v[j, pl.ds(i * 16, 16)]
            d16 = dst_v[j, pl.ds(i * 16, 16)]
            src2_v[j, pl.ds(i * 16, 16)] = s16 + off
            av = plsc.load_gather(asr_v, [s16])
            bv = plsc.load_gather(adst_v, [d16])
            x = av + bv
            al = jnp.maximum(x, 0.2 * x)
            ew_v[j, pl.ds(i * 16, 16)] = al
            cmax = jnp.maximum(cmax, al)
        return cmax

    with jax.named_scope("edge_p1"):
        cmax = lax.fori_loop(0, NBLK, p1,
                             jnp.full((16,), -1e30, jnp.float32))
    st_v[...] = jnp.full((16,), jnp.max(cmax), jnp.float32)
    pltpu.sync_copy(st_v, mx_sh.at[sid])
    plsc.subcore_barrier()

    pltpu.sync_copy(mx_sh, mxv_v)
    mv = mxv_v[0]
    for i in range(1, 16):
        mv = jnp.maximum(mv, mxv_v[i])

    def g_start(j, b):
        pltpu.async_copy(ksrc_hbm.at[src2_v.at[j]], rows_v.at[b], sem)

    def g_wait(j, b):
        pltpu.make_async_copy(ksrc_hbm.at[src2_v.at[j]], rows_v.at[b],
                              sem).wait()

    def s_start(j, b):
        pltpu.async_copy(rows_v.at[b], acc_sh.at[dst_v.at[j]], sem_s,
                         add=True)
        pltpu.async_copy(w_v.at[b], s_sh.at[dst_v.at[j]], sem_s, add=True)

    def s_wait(j, b):
        pltpu.make_async_copy(rows_v.at[b], acc_sh.at[dst_v.at[j]],
                              sem_s).wait()
        pltpu.make_async_copy(w_v.at[b], s_sh.at[dst_v.at[j]],
                              sem_s).wait()

    g_start(0, 0)

    def p3(jj, carry):
        for b in range(2):
            j = jj * 2 + b
            g_wait(j, b)

            @pl.when(j + 1 < NBLK)
            def _():
                @pl.when(j >= 1)
                def _():
                    s_wait(j - 1, 1 - b)
                g_start(j + 1, 1 - b)

            for i in range(8):
                al = ew_v[j, pl.ds(i * 16, 16)]
                w_v[b, pl.ds(i * 16, 16)] = jnp.exp(al - mv)

            @plsc.parallel_loop(0, BLK)
            def _(r):
                wr = plsc.load_gather(
                    w_v, [jnp.full((16,), b, jnp.int32),
                          jnp.full((16,), 0, jnp.int32) + r])
                for c in range(8):
                    rv = rows_v[b, r, pl.ds(c * 16, 16)]
                    rows_v[b, r, pl.ds(c * 16, 16)] = rv * wr

            s_start(j, b)
        return carry

    with jax.named_scope("edge_p3"):
        lax.fori_loop(0, NBLK // 2, p3, 0)
        s_wait(NBLK - 2, 0)
        s_wait(NBLK - 1, 1)
    plsc.subcore_barrier()

    with jax.named_scope("edge_out"):
        pltpu.sync_copy(s_sh, sden_v)
        base = sid * ROWS_PER_TILE
        for off, sz in ((0, BLK), (BLK, BLK), (2 * BLK, ROWS_PER_TILE - 2 * BLK)):
            pltpu.sync_copy(acc_sh.at[pl.ds(base + off, sz)],
                            rows_v.at[0, pl.ds(0, sz)])

            @plsc.parallel_loop(0, sz)
            def _(r):
                sv = plsc.load_gather(
                    sden_v, [jnp.full((16,), base + off, jnp.int32) + r])
                inv = 1.0 / (sv + 1e-16)
                for c in range(8):
                    rv = rows_v[0, r, pl.ds(c * 16, 16)]
                    rows_v[0, r, pl.ds(c * 16, 16)] = rv * inv

            pltpu.sync_copy(rows_v.at[0, pl.ds(0, sz)],
                            out_hbm.at[cid, pl.ds(base + off, sz)])


_edge_call = pl.kernel(
    _edge_body,
    out_type=jax.ShapeDtypeStruct((2, NPAD, HHALF), jnp.float32),
    mesh=plsc.VectorSubcoreMesh(core_axis_name="c", subcore_axis_name="s"),
    compiler_params=pltpu.CompilerParams(needs_layout_passes=False),
    scratch_types=[
        pltpu.VMEM((NBLK, BLK), jnp.int32),
        pltpu.VMEM((NBLK, BLK), jnp.int32),
        pltpu.VMEM((NBLK, BLK), jnp.int32),
        pltpu.VMEM((NPAD,), jnp.float32),
        pltpu.VMEM((NPAD,), jnp.float32),
        pltpu.VMEM((NBLK, BLK), jnp.float32),
        pltpu.VMEM((NPAD,), jnp.float32),
        pltpu.VMEM((2, BLK), jnp.float32),
        pltpu.VMEM((2, BLK, HHALF), jnp.float32),
        pltpu.VMEM((64, HHALF), jnp.float32),
        pltpu.VMEM((16, 16), jnp.float32),
        pltpu.VMEM((16,), jnp.float32),
        pltpu.VMEM_SHARED((NPAD, HHALF), jnp.float32),
        pltpu.VMEM_SHARED((NPAD,), jnp.float32),
        pltpu.VMEM_SHARED((16, 16), jnp.float32),
        pltpu.SemaphoreType.DMA,
        pltpu.SemaphoreType.DMA,
    ],
)


def _pad_edges(ei):
    npad = EPAD - ei.shape[1]
    fill = N_REAL + (jnp.arange(npad, dtype=jnp.int32) % (NPAD - N_REAL))
    s = jnp.concatenate([ei[0], fill])
    d = jnp.concatenate([ei[1], fill])
    return s.reshape(16 * NBLK, BLK), d.reshape(16 * NBLK, BLK)


def _split_k(k):
    return k.reshape(NPAD, 2, HHALF).transpose(1, 0, 2).reshape(2 * NPAD, HHALF)


def _amt(att_src_vec, att_dst_vec):
    z = jnp.zeros((6, HID), jnp.float32)
    return jnp.concatenate([att_src_vec[None], att_dst_vec[None], z], axis=0)


def kernel(x_author, x_paper, edge_index_writes, edge_index_rev, params):
    p = params
    xa = jnp.pad(x_author, ((0, NPAD - N_REAL), (0, 0)))
    xp = jnp.pad(x_paper, ((0, NPAD - N_REAL), (0, 0)))
    sw, dw = _pad_edges(edge_index_writes)
    sr, dr = _pad_edges(edge_index_rev)

    c1, c2 = p['conv1'], p['conv2']
    w1a = p['Win']['author'] @ c1['proj_W']['author']
    b1a = p['bin']['author'] @ c1['proj_W']['author'] + c1['proj_b']['author']
    w1p = p['Win']['paper'] @ c1['proj_W']['paper']
    b1p = p['bin']['paper'] @ c1['proj_W']['paper'] + c1['proj_b']['paper']

    amt_a1 = _amt(c1['att_src']['writes'], c1['att_dst']['rev'])
    amt_p1 = _amt(c1['att_src']['rev'], c1['att_dst']['writes'])
    ka1, ga1 = _kg_full(xa, w1a, b1a[None], amt_a1)
    kp1, gp1 = _kg_full(xp, w1p, b1p[None], amt_p1)
    op1 = _edge_call(_split_k(ka1), ga1, gp1, sw, dw)
    oa1 = _edge_call(_split_k(kp1), gp1, ga1, sr, dr)

    amt_a2 = _amt(c2['att_src']['writes'], c2['att_dst']['rev'])
    amt_p2 = _amt(c2['att_src']['rev'], c2['att_dst']['writes'])
    ka2, ga2 = _kg_halves(oa1[0], oa1[1], c2['proj_W']['author'],
                          c2['proj_b']['author'][None], amt_a2)
    kp2, gp2 = _kg_halves(op1[0], op1[1], c2['proj_W']['paper'],
                          c2['proj_b']['paper'][None], amt_p2)
    op2 = _edge_call(_split_k(ka2), ga2, gp2, sw, dw)
    oa2 = _edge_call(_split_k(kp2), gp2, ga2, sr, dr)

    out_a = _fin(oa2[0], oa2[1], p['Wout'], p['bout'][None])
    out_p = _fin(op2[0], op2[1], p['Wout'], p['bout'][None])
    return jnp.concatenate([out_a[:N_REAL], out_p[:N_REAL]], axis=0)


# speedup vs baseline: 23.9814x; 1.0875x over previous
import jax
import jax.numpy as jnp
from jax import lax
from jax.experimental import pallas as pl
from jax.experimental.pallas import tpu as pltpu
from jax.experimental.pallas import tpu_sc as plsc

HID = 256
HHALF = 128
N_REAL = 5000
NPAD = 5120
E_REAL = 80000
EPAD = 81920
NBLK = 80
BLK = 64
NBUF = 4
ROWS_PER_TILE = NPAD // 16


_BM = 512


def _fin_body(xl_ref, xh_ref, w_ref, b_ref, o_ref):
    x = jnp.concatenate([xl_ref[...], xh_ref[...]], axis=1)
    o_ref[...] = (
        jnp.dot(x, w_ref[...], preferred_element_type=jnp.float32) + b_ref[...]
    )


def _fin(xl, xh, w, b):
    m = xl.shape[0]
    hspec = pl.BlockSpec((_BM, HHALF), lambda i: (i, 0))
    return pl.pallas_call(
        _fin_body,
        grid=(m // _BM,),
        in_specs=[
            hspec,
            hspec,
            pl.BlockSpec((HID, HID), lambda i: (0, 0)),
            pl.BlockSpec((1, HID), lambda i: (0, 0)),
        ],
        out_specs=pl.BlockSpec((_BM, HID), lambda i: (i, 0)),
        out_shape=jax.ShapeDtypeStruct((m, HID), jnp.float32),
    )(xl, xh, w, b)


def _kg_post(x, w_ref, b_ref, amt_ref, k_ref, g_ref):
    k = jnp.dot(x, w_ref[...], preferred_element_type=jnp.float32) + b_ref[...]
    k_ref[0] = k[:, :HHALF]
    k_ref[1] = k[:, HHALF:]
    g_ref[...] = lax.dot_general(
        amt_ref[...], k, (((1,), (1,)), ((), ())),
        preferred_element_type=jnp.float32,
    )


def _kg_full_body(x_ref, w_ref, b_ref, amt_ref, k_ref, g_ref):
    _kg_post(x_ref[...], w_ref, b_ref, amt_ref, k_ref, g_ref)


def _kg_halves_body(xl_ref, xh_ref, w_ref, b_ref, amt_ref, k_ref, g_ref):
    x = jnp.concatenate([xl_ref[...], xh_ref[...]], axis=1)
    x = jnp.maximum(x, 0.01 * x)
    _kg_post(x, w_ref, b_ref, amt_ref, k_ref, g_ref)


def _kg_call(body, in_specs, args, m):
    return pl.pallas_call(
        body,
        grid=(m // _BM,),
        in_specs=in_specs + [
            pl.BlockSpec((HID, HID), lambda i: (0, 0)),
            pl.BlockSpec((1, HID), lambda i: (0, 0)),
            pl.BlockSpec((8, HID), lambda i: (0, 0)),
        ],
        out_specs=[
            pl.BlockSpec((2, _BM, HHALF), lambda i: (0, i, 0)),
            pl.BlockSpec((8, _BM), lambda i: (0, i)),
        ],
        out_shape=[
            jax.ShapeDtypeStruct((2, m, HHALF), jnp.float32),
            jax.ShapeDtypeStruct((8, m), jnp.float32),
        ],
    )(*args)


def _kg_full(x, w, b, amt):
    return _kg_call(
        _kg_full_body,
        [pl.BlockSpec((_BM, HID), lambda i: (i, 0))],
        [x, w, b, amt], x.shape[0])


def _kg_halves(xl, xh, w, b, amt):
    hspec = pl.BlockSpec((_BM, HHALF), lambda i: (i, 0))
    return _kg_call(
        _kg_halves_body, [hspec, hspec], [xl, xh, w, b, amt], xl.shape[0])


def _edge_body(ksrc_hbm, gsrc_hbm, gdst_hbm, srcp_hbm, dstp_hbm,
               out_hbm,
               dst_v, src2_v, asr_v, adst_v, ew_v, sden_v, w_v, rows_v,
               zb_v, mxv_v, st_v, acc_sh, s_sh, mx_sh, sem, sem_s):
    cid = lax.axis_index("c")
    sid = lax.axis_index("s")

    pltpu.async_copy(srcp_hbm.at[pl.ds(sid * NBLK, NBLK)], src2_v, sem)
    pltpu.async_copy(dstp_hbm.at[pl.ds(sid * NBLK, NBLK)], dst_v, sem)
    pltpu.async_copy(gsrc_hbm.at[0], asr_v, sem)
    pltpu.async_copy(gdst_hbm.at[1], adst_v, sem)

    zv = jnp.zeros((16,), jnp.float32)

    def zb(i, c2):
        for k in range(8):
            zb_v[i, pl.ds(k * 16, 16)] = zv
        return c2

    lax.fori_loop(0, 16, zb, 0)
    for t in range(ROWS_PER_TILE // 16):
        pltpu.async_copy(
            zb_v, acc_sh.at[pl.ds(sid * ROWS_PER_TILE + t * 16, 16)], sem_s)

    pltpu.make_async_copy(srcp_hbm.at[pl.ds(0, NBLK)], src2_v, sem).wait()
    pltpu.make_async_copy(dstp_hbm.at[pl.ds(0, NBLK)], dst_v, sem).wait()
    pltpu.make_async_copy(gsrc_hbm.at[0], asr_v, sem).wait()
    pltpu.make_async_copy(gdst_hbm.at[1], adst_v, sem).wait()
    for t in range(ROWS_PER_TILE // 16):
        pltpu.make_async_copy(
            zb_v, acc_sh.at[pl.ds(sid * ROWS_PER_TILE + t * 16, 16)],
            sem_s).wait()

    def zs(i, c2):
        sden_v[pl.ds(i * 16, 16)] = zv
        return c2

    lax.fori_loop(0, NPAD // 16, zs, 0)

    @pl.when(sid == 0)
    def _():
        pltpu.sync_copy(sden_v, s_sh)

    off = (cid * NPAD).astype(jnp.int32)

    def p1(j, cmax):
        for i in range(BLK // 16):
            s16 = src2_v[j, pl.ds(i * 16, 16)]
            d16 = dst_v[j, pl.ds(i * 16, 16)]
            src2_v[j, pl.ds(i * 16, 16)] = s16 + off
            av = plsc.load_gather(asr_v, [s16])
            bv = plsc.load_gather(adst_v, [d16])
            x = av + bv
            al = jnp.maximum(x, 0.2 * x)
            ew_v[j, pl.ds(i * 16, 16)] = al
            cmax = jnp.maximum(cmax, al)
        return cmax

    with jax.named_scope("edge_p1"):
        cmax = lax.fori_loop(0, NBLK, p1,
                             jnp.full((16,), -1e30, jnp.float32))
    st_v[...] = jnp.full((16,), jnp.max(cmax), jnp.float32)
    pltpu.sync_copy(st_v, mx_sh.at[sid])
    plsc.subcore_barrier()

    pltpu.sync_copy(mx_sh, mxv_v)
    mv = mxv_v[0]
    for i in range(1, 16):
        mv = jnp.maximum(mv, mxv_v[i])

    def g_start(j, b):
        pltpu.async_copy(ksrc_hbm.at[src2_v.at[j]], rows_v.at[b], sem)

    def g_wait(j, b):
        pltpu.make_async_copy(ksrc_hbm.at[src2_v.at[j]], rows_v.at[b],
                              sem).wait()

    def s_start(j, b):
        pltpu.async_copy(rows_v.at[b], acc_sh.at[dst_v.at[j]], sem_s,
                         add=True)
        pltpu.async_copy(w_v.at[b], s_sh.at[dst_v.at[j]], sem_s, add=True)

    def s_wait(j, b):
        pltpu.make_async_copy(rows_v.at[b], acc_sh.at[dst_v.at[j]],
                              sem_s).wait()
        pltpu.make_async_copy(w_v.at[b], s_sh.at[dst_v.at[j]],
                              sem_s).wait()

    g_start(0, 0)
    g_start(1, 1)

    def p3(jj, carry):
        for b in range(NBUF):
            j = jj * NBUF + b
            g_wait(j, b)
            bn = (b + 2) % NBUF

            @pl.when(j + 2 < NBLK)
            def _():
                @pl.when(j >= 2)
                def _():
                    s_wait(j - 2, bn)
                g_start(j + 2, bn)

            for i in range(BLK // 16):
                al = ew_v[j, pl.ds(i * 16, 16)]
                w_v[b, pl.ds(i * 16, 16)] = jnp.exp(al - mv)

            @plsc.parallel_loop(0, BLK)
            def _(r):
                wr = plsc.load_gather(
                    w_v, [jnp.full((16,), b, jnp.int32),
                          jnp.full((16,), 0, jnp.int32) + r])
                for c in range(8):
                    rv = rows_v[b, r, pl.ds(c * 16, 16)]
                    rows_v[b, r, pl.ds(c * 16, 16)] = rv * wr

            s_start(j, b)
        return carry

    with jax.named_scope("edge_p3"):
        lax.fori_loop(0, NBLK // NBUF, p3, 0)
        for b in range(NBUF):
            s_wait(NBLK - NBUF + b, b)
    plsc.subcore_barrier()

    with jax.named_scope("edge_out"):
        pltpu.sync_copy(s_sh, sden_v)
        base = sid * ROWS_PER_TILE
        for off, sz in [(q * BLK, BLK) for q in range(ROWS_PER_TILE // BLK)]:
            pltpu.sync_copy(acc_sh.at[pl.ds(base + off, sz)],
                            rows_v.at[0, pl.ds(0, sz)])

            @plsc.parallel_loop(0, sz)
            def _(r):
                sv = plsc.load_gather(
                    sden_v, [jnp.full((16,), base + off, jnp.int32) + r])
                inv = 1.0 / (sv + 1e-16)
                for c in range(8):
                    rv = rows_v[0, r, pl.ds(c * 16, 16)]
                    rows_v[0, r, pl.ds(c * 16, 16)] = rv * inv

            pltpu.sync_copy(rows_v.at[0, pl.ds(0, sz)],
                            out_hbm.at[cid, pl.ds(base + off, sz)])


_edge_call = pl.kernel(
    _edge_body,
    out_type=jax.ShapeDtypeStruct((2, NPAD, HHALF), jnp.float32),
    mesh=plsc.VectorSubcoreMesh(core_axis_name="c", subcore_axis_name="s"),
    compiler_params=pltpu.CompilerParams(needs_layout_passes=False),
    scratch_types=[
        pltpu.VMEM((NBLK, BLK), jnp.int32),
        pltpu.VMEM((NBLK, BLK), jnp.int32),
        pltpu.VMEM((NPAD,), jnp.float32),
        pltpu.VMEM((NPAD,), jnp.float32),
        pltpu.VMEM((NBLK, BLK), jnp.float32),
        pltpu.VMEM((NPAD,), jnp.float32),
        pltpu.VMEM((NBUF, BLK), jnp.float32),
        pltpu.VMEM((NBUF, BLK, HHALF), jnp.float32),
        pltpu.VMEM((16, HHALF), jnp.float32),
        pltpu.VMEM((16, 16), jnp.float32),
        pltpu.VMEM((16,), jnp.float32),
        pltpu.VMEM_SHARED((NPAD, HHALF), jnp.float32),
        pltpu.VMEM_SHARED((NPAD,), jnp.float32),
        pltpu.VMEM_SHARED((16, 16), jnp.float32),
        pltpu.SemaphoreType.DMA,
        pltpu.SemaphoreType.DMA,
    ],
)


def _pad_edges(ei):
    npad = EPAD - ei.shape[1]
    fill = N_REAL + (jnp.arange(npad, dtype=jnp.int32) % (NPAD - N_REAL))
    s = jnp.concatenate([ei[0], fill])
    d = jnp.concatenate([ei[1], fill])
    return s.reshape(16 * NBLK, BLK), d.reshape(16 * NBLK, BLK)


def _split_k(k):
    return k.reshape(2 * NPAD, HHALF)


def _amt(att_src_vec, att_dst_vec):
    z = jnp.zeros((6, HID), jnp.float32)
    return jnp.concatenate([att_src_vec[None], att_dst_vec[None], z], axis=0)


def kernel(x_author, x_paper, edge_index_writes, edge_index_rev, params):
    p = params
    xa = jnp.pad(x_author, ((0, NPAD - N_REAL), (0, 0)))
    xp = jnp.pad(x_paper, ((0, NPAD - N_REAL), (0, 0)))
    sw, dw = _pad_edges(edge_index_writes)
    sr, dr = _pad_edges(edge_index_rev)

    c1, c2 = p['conv1'], p['conv2']
    w1a = p['Win']['author'] @ c1['proj_W']['author']
    b1a = p['bin']['author'] @ c1['proj_W']['author'] + c1['proj_b']['author']
    w1p = p['Win']['paper'] @ c1['proj_W']['paper']
    b1p = p['bin']['paper'] @ c1['proj_W']['paper'] + c1['proj_b']['paper']

    amt_a1 = _amt(c1['att_src']['writes'], c1['att_dst']['rev'])
    amt_p1 = _amt(c1['att_src']['rev'], c1['att_dst']['writes'])
    ka1, ga1 = _kg_full(xa, w1a, b1a[None], amt_a1)
    kp1, gp1 = _kg_full(xp, w1p, b1p[None], amt_p1)
    op1 = _edge_call(_split_k(ka1), ga1, gp1, sw, dw)
    oa1 = _edge_call(_split_k(kp1), gp1, ga1, sr, dr)

    amt_a2 = _amt(c2['att_src']['writes'], c2['att_dst']['rev'])
    amt_p2 = _amt(c2['att_src']['rev'], c2['att_dst']['writes'])
    ka2, ga2 = _kg_halves(oa1[0], oa1[1], c2['proj_W']['author'],
                          c2['proj_b']['author'][None], amt_a2)
    kp2, gp2 = _kg_halves(op1[0], op1[1], c2['proj_W']['paper'],
                          c2['proj_b']['paper'][None], amt_p2)
    op2 = _edge_call(_split_k(ka2), ga2, gp2, sw, dw)
    oa2 = _edge_call(_split_k(kp2), gp2, ga2, sr, dr)

    out_a = _fin(oa2[0], oa2[1], p['Wout'], p['bout'][None])
    out_p = _fin(op2[0], op2[1], p['Wout'], p['bout'][None])
    return jnp.concatenate([out_a[:N_REAL], out_p[:N_REAL]], axis=0)
